# Initial kernel scaffold; baseline (speedup 1.0000x reference)
import jax
import jax.numpy as jnp
from jax.experimental import pallas as pl


def kernel(x, edge_index, W1, b1, W2, b2, W3, b3, g1, be1, g2, be2, g3, be3, p, Wg, bg):
    raise NotImplementedError("write your pallas kernel here")


# speedup vs baseline: 2.9728x; 2.9728x over previous
import functools

import jax
import jax.numpy as jnp
from jax import lax
from jax.experimental import pallas as pl
from jax.experimental.pallas import tpu as pltpu
from jax.experimental.pallas import tpu_sc as plsc

N_NODES = 10000
N_EDGES = 320000
D = 128
N_CLASS = 10

NUM_CORES = 2
NUM_SUBCORES = 16
NUM_TILES = NUM_CORES * NUM_SUBCORES

CHUNK = 128
EDGE_ROWS = 2560
ROWS_PER_TILE = EDGE_ROWS // NUM_TILES
AGG_ROWS = 10240
ROWS_PER_SUBCORE = AGG_ROWS // NUM_SUBCORES
DUMMY_ROW = N_NODES


def _sc_segment_sum(h, src2d, dst2d):
    mesh = plsc.VectorSubcoreMesh(core_axis_name="c", subcore_axis_name="s")

    @functools.partial(
        pl.kernel,
        mesh=mesh,
        out_type=jax.ShapeDtypeStruct((NUM_CORES, AGG_ROWS, D), jnp.float32),
        scratch_types=[
            pltpu.VMEM((ROWS_PER_TILE, CHUNK), jnp.int32),
            pltpu.VMEM((ROWS_PER_TILE, CHUNK), jnp.int32),
            pltpu.VMEM((CHUNK, D), jnp.float32),
            pltpu.VMEM_SHARED((AGG_ROWS, D), jnp.float32),
            pltpu.SemaphoreType.DMA,
        ],
    )
    def k(h_hbm, src_hbm, dst_hbm, out_hbm, src_v, dst_v, rows_v, agg_sh, sem):
        c = lax.axis_index("c")
        s = lax.axis_index("s")
        tid = c * NUM_SUBCORES + s

        def zrow(i, carry):
            def zcol(j, carry2):
                rows_v[i, pl.ds(j * 16, 16)] = jnp.zeros((16,), jnp.float32)
                return carry2
            return lax.fori_loop(0, D // 16, zcol, carry)
        lax.fori_loop(0, CHUNK, zrow, 0)
        zbase = s * ROWS_PER_SUBCORE
        for z in range(ROWS_PER_SUBCORE // CHUNK):
            pltpu.sync_copy(rows_v, agg_sh.at[pl.ds(zbase + z * CHUNK, CHUNK)])
        plsc.subcore_barrier()

        pltpu.sync_copy(src_hbm.at[pl.ds(tid * ROWS_PER_TILE, ROWS_PER_TILE)], src_v)
        pltpu.sync_copy(dst_hbm.at[pl.ds(tid * ROWS_PER_TILE, ROWS_PER_TILE)], dst_v)

        def body(j, carry):
            pltpu.async_copy(h_hbm.at[src_v.at[j]], rows_v, sem).wait()
            pltpu.sync_copy(rows_v, agg_sh.at[dst_v.at[j]], add=True)
            return carry
        lax.fori_loop(0, ROWS_PER_TILE, body, 0)
        plsc.subcore_barrier()

        obase = s * ROWS_PER_SUBCORE
        pltpu.sync_copy(agg_sh.at[pl.ds(obase, ROWS_PER_SUBCORE)],
                        out_hbm.at[c, pl.ds(obase, ROWS_PER_SUBCORE)])

    return k(h, src2d, dst2d)


def _tc_linear(x, Wt, b2d):
    def k(x_ref, w_ref, b_ref, o_ref):
        o_ref[...] = jnp.dot(x_ref[...], w_ref[...],
                             preferred_element_type=jnp.float32) + b_ref[...]
    return pl.pallas_call(
        k, out_shape=jax.ShapeDtypeStruct((N_NODES, D), jnp.float32),
    )(x, Wt, b2d)


def _combine_bn_leaky(p_ref, h_ref, g_ref, be_ref):
    t = p_ref[0, :N_NODES] + p_ref[1, :N_NODES] + h_ref[...]
    t = jnp.maximum(t, 0.0)
    mu = jnp.mean(t, axis=0, keepdims=True)
    var = jnp.mean((t - mu) * (t - mu), axis=0, keepdims=True)
    tn = g_ref[...] * (t - mu) / jnp.sqrt(var + 1e-5) + be_ref[...]
    return jnp.where(tn >= 0.0, tn, 0.1 * tn)


def _tc_mid(P, h, g2d, be2d, Wt, b2d):
    def k(p_ref, h_ref, g_ref, be_ref, w_ref, b_ref, o_ref):
        tl = _combine_bn_leaky(p_ref, h_ref, g_ref, be_ref)
        o_ref[...] = jnp.dot(tl, w_ref[...],
                             preferred_element_type=jnp.float32) + b_ref[...]
    return pl.pallas_call(
        k, out_shape=jax.ShapeDtypeStruct((N_NODES, D), jnp.float32),
    )(P, h, g2d, be2d, Wt, b2d)


def _tc_final(P, h, g2d, be2d, p2d, WgT, bg2d):
    def k(p_ref, h_ref, g_ref, be_ref, pw_ref, wg_ref, bg_ref, out_ref, yp_ref):
        tl = _combine_bn_leaky(p_ref, h_ref, g_ref, be_ref)
        pw = pw_ref[0, 0]
        xc = jnp.clip(tl, 0.0, 100.0)
        xp = jnp.exp(pw * jnp.log(xc))
        pool = jnp.mean(xp, axis=0, keepdims=True)
        pool = jnp.clip(pool, 0.0, 100.0)
        pool = jnp.exp(jnp.log(pool) / pw)
        logits = jnp.dot(pool, wg_ref[...],
                         preferred_element_type=jnp.float32) + bg_ref[...]
        out_ref[...] = logits
        mx = jnp.max(logits, axis=1, keepdims=True)
        ids = lax.broadcasted_iota(jnp.int32, (1, N_CLASS), 1)
        cand = jnp.where(logits >= mx, ids, N_CLASS)
        yp_ref[...] = jnp.min(cand, axis=1, keepdims=True)
    return pl.pallas_call(
        k,
        out_shape=(jax.ShapeDtypeStruct((1, N_CLASS), jnp.float32),
                   jax.ShapeDtypeStruct((1, 1), jnp.int32)),
    )(P, h, g2d, be2d, p2d, WgT, bg2d)


def kernel(x, edge_index, W1, b1, W2, b2, W3, b3, g1, be1, g2, be2, g3, be3, p, Wg, bg):
    ei = edge_index.astype(jnp.int32)
    pad = EDGE_ROWS * CHUNK - N_EDGES
    src2d = jnp.concatenate(
        [ei[0], jnp.zeros((pad,), jnp.int32)]).reshape(EDGE_ROWS, CHUNK)
    dst2d = jnp.concatenate(
        [ei[1], jnp.full((pad,), DUMMY_ROW, jnp.int32)]).reshape(EDGE_ROWS, CHUNK)

    W1t, W2t, W3t, WgT = W1.T, W2.T, W3.T, Wg.T
    b1d, b2d, b3d = b1.reshape(1, D), b2.reshape(1, D), b3.reshape(1, D)
    g1d, g2_2d, g3d = g1.reshape(1, D), g2.reshape(1, D), g3.reshape(1, D)
    be1d, be2d, be3d = be1.reshape(1, D), be2.reshape(1, D), be3.reshape(1, D)
    bg2d = bg.reshape(1, N_CLASS)
    p2d = p.reshape(1, 1)

    h1 = _tc_linear(x, W1t, b1d)
    P1 = _sc_segment_sum(h1, src2d, dst2d)
    h2 = _tc_mid(P1, h1, g1d, be1d, W2t, b2d)
    P2 = _sc_segment_sum(h2, src2d, dst2d)
    h3 = _tc_mid(P2, h2, g2_2d, be2d, W3t, b3d)
    P3 = _sc_segment_sum(h3, src2d, dst2d)
    output, yp = _tc_final(P3, h3, g3d, be3d, p2d, WgT, bg2d)
    return (output, yp.reshape(1))


# speedup vs baseline: 3.0306x; 1.0195x over previous
import functools

import jax
import jax.numpy as jnp
from jax import lax
from jax.experimental import pallas as pl
from jax.experimental.pallas import tpu as pltpu
from jax.experimental.pallas import tpu_sc as plsc

N_NODES = 10000
N_EDGES = 320000
D = 128
N_CLASS = 10

NUM_CORES = 2
NUM_SUBCORES = 16
NUM_TILES = NUM_CORES * NUM_SUBCORES

CHUNK = 128
EDGE_ROWS = 2560
ROWS_PER_TILE = EDGE_ROWS // NUM_TILES
NBUF = 2
IDX_SEG = 40
AGG_ROWS = 10240
ROWS_PER_SUBCORE = AGG_ROWS // NUM_SUBCORES
DUMMY_ROW = N_NODES


def _sc_segment_sum(h, src2d, dst2d):
    mesh = plsc.VectorSubcoreMesh(core_axis_name="c", subcore_axis_name="s")

    @functools.partial(
        pl.kernel,
        mesh=mesh,
        out_type=jax.ShapeDtypeStruct((NUM_CORES, AGG_ROWS, D), jnp.float32),
        scratch_types=[
            pltpu.VMEM((IDX_SEG, CHUNK), jnp.int32),
            pltpu.VMEM((IDX_SEG, CHUNK), jnp.int32),
            pltpu.VMEM((NBUF, CHUNK, D), jnp.float32),
            pltpu.VMEM_SHARED((AGG_ROWS, D), jnp.float32),
            pltpu.SemaphoreType.DMA,
            pltpu.SemaphoreType.DMA,
        ],
    )
    def k(h_hbm, src_hbm, dst_hbm, out_hbm, src_v, dst_v, rows_v, agg_sh,
          sem0, sem1):
        c = lax.axis_index("c")
        s = lax.axis_index("s")
        tid = c * NUM_SUBCORES + s

        def zrow(i, carry):
            def zcol(j, carry2):
                rows_v[0, i, pl.ds(j * 16, 16)] = jnp.zeros((16,), jnp.float32)
                return carry2
            return lax.fori_loop(0, D // 16, zcol, carry)
        lax.fori_loop(0, CHUNK, zrow, 0)
        zbase = s * ROWS_PER_SUBCORE
        for z in range(ROWS_PER_SUBCORE // CHUNK):
            pltpu.sync_copy(rows_v.at[0], agg_sh.at[pl.ds(zbase + z * CHUNK, CHUNK)])
        plsc.subcore_barrier()

        sems = [sem0, sem1]
        for seg in range(ROWS_PER_TILE // IDX_SEG):
            ibase = tid * ROWS_PER_TILE + seg * IDX_SEG
            pltpu.sync_copy(src_hbm.at[pl.ds(ibase, IDX_SEG)], src_v)
            pltpu.sync_copy(dst_hbm.at[pl.ds(ibase, IDX_SEG)], dst_v)

            def body(i, carry):
                j = i * NBUF
                cds = [pltpu.async_copy(h_hbm.at[src_v.at[j + b]], rows_v.at[b],
                                        sems[b])
                       for b in range(NBUF)]
                for b in range(NBUF):
                    cds[b].wait()
                    pltpu.sync_copy(rows_v.at[b], agg_sh.at[dst_v.at[j + b]],
                                    add=True)
                return carry
            lax.fori_loop(0, IDX_SEG // NBUF, body, 0)
        plsc.subcore_barrier()

        obase = s * ROWS_PER_SUBCORE
        pltpu.sync_copy(agg_sh.at[pl.ds(obase, ROWS_PER_SUBCORE)],
                        out_hbm.at[c, pl.ds(obase, ROWS_PER_SUBCORE)])

    return k(h, src2d, dst2d)


def _tc_linear(x, Wt, b2d):
    def k(x_ref, w_ref, b_ref, o_ref):
        o_ref[...] = jnp.dot(x_ref[...], w_ref[...],
                             preferred_element_type=jnp.float32) + b_ref[...]
    return pl.pallas_call(
        k, out_shape=jax.ShapeDtypeStruct((N_NODES, D), jnp.float32),
    )(x, Wt, b2d)


def _combine_bn_leaky(p_ref, h_ref, g_ref, be_ref):
    t = p_ref[0, :N_NODES] + p_ref[1, :N_NODES] + h_ref[...]
    t = jnp.maximum(t, 0.0)
    mu = jnp.mean(t, axis=0, keepdims=True)
    var = jnp.mean((t - mu) * (t - mu), axis=0, keepdims=True)
    tn = g_ref[...] * (t - mu) / jnp.sqrt(var + 1e-5) + be_ref[...]
    return jnp.where(tn >= 0.0, tn, 0.1 * tn)


def _tc_mid(P, h, g2d, be2d, Wt, b2d):
    def k(p_ref, h_ref, g_ref, be_ref, w_ref, b_ref, o_ref):
        tl = _combine_bn_leaky(p_ref, h_ref, g_ref, be_ref)
        o_ref[...] = jnp.dot(tl, w_ref[...],
                             preferred_element_type=jnp.float32) + b_ref[...]
    return pl.pallas_call(
        k, out_shape=jax.ShapeDtypeStruct((N_NODES, D), jnp.float32),
    )(P, h, g2d, be2d, Wt, b2d)


def _tc_final(P, h, g2d, be2d, p2d, WgT, bg2d):
    def k(p_ref, h_ref, g_ref, be_ref, pw_ref, wg_ref, bg_ref, out_ref, yp_ref):
        tl = _combine_bn_leaky(p_ref, h_ref, g_ref, be_ref)
        pw = pw_ref[0, 0]
        xc = jnp.clip(tl, 0.0, 100.0)
        xp = jnp.exp(pw * jnp.log(xc))
        pool = jnp.mean(xp, axis=0, keepdims=True)
        pool = jnp.clip(pool, 0.0, 100.0)
        pool = jnp.exp(jnp.log(pool) / pw)
        logits = jnp.dot(pool, wg_ref[...],
                         preferred_element_type=jnp.float32) + bg_ref[...]
        out_ref[...] = logits
        mx = jnp.max(logits, axis=1, keepdims=True)
        ids = lax.broadcasted_iota(jnp.int32, (1, N_CLASS), 1)
        cand = jnp.where(logits >= mx, ids, N_CLASS)
        yp_ref[...] = jnp.min(cand, axis=1, keepdims=True)
    return pl.pallas_call(
        k,
        out_shape=(jax.ShapeDtypeStruct((1, N_CLASS), jnp.float32),
                   jax.ShapeDtypeStruct((1, 1), jnp.int32)),
    )(P, h, g2d, be2d, p2d, WgT, bg2d)


def kernel(x, edge_index, W1, b1, W2, b2, W3, b3, g1, be1, g2, be2, g3, be3, p, Wg, bg):
    ei = edge_index.astype(jnp.int32)
    pad = EDGE_ROWS * CHUNK - N_EDGES
    src2d = jnp.concatenate(
        [ei[0], jnp.zeros((pad,), jnp.int32)]).reshape(EDGE_ROWS, CHUNK)
    dst2d = jnp.concatenate(
        [ei[1], jnp.full((pad,), DUMMY_ROW, jnp.int32)]).reshape(EDGE_ROWS, CHUNK)

    W1t, W2t, W3t, WgT = W1.T, W2.T, W3.T, Wg.T
    b1d, b2d, b3d = b1.reshape(1, D), b2.reshape(1, D), b3.reshape(1, D)
    g1d, g2_2d, g3d = g1.reshape(1, D), g2.reshape(1, D), g3.reshape(1, D)
    be1d, be2d, be3d = be1.reshape(1, D), be2.reshape(1, D), be3.reshape(1, D)
    bg2d = bg.reshape(1, N_CLASS)
    p2d = p.reshape(1, 1)

    h1 = _tc_linear(x, W1t, b1d)
    P1 = _sc_segment_sum(h1, src2d, dst2d)
    h2 = _tc_mid(P1, h1, g1d, be1d, W2t, b2d)
    P2 = _sc_segment_sum(h2, src2d, dst2d)
    h3 = _tc_mid(P2, h2, g2_2d, be2d, W3t, b3d)
    P3 = _sc_segment_sum(h3, src2d, dst2d)
    output, yp = _tc_final(P3, h3, g3d, be3d, p2d, WgT, bg2d)
    return (output, yp.reshape(1))
